# baseline (speedup 1.0000x reference)
import functools

import jax
import jax.numpy as jnp
from jax import lax
from jax.experimental import pallas as pl
from jax.experimental.pallas import tpu as pltpu
from jax.experimental.pallas import tpu_sc as plsc

N = 10000
E = 320000
D = 128
H = 128
G = 256
EMB = 300
C = 10

NC = 2
NS = 16
NW = NC * NS
CHUNK = 128
NCHUNK = 80
HALF = NCHUNK // 2
EP = NW * NCHUNK * CHUNK
EPT = NCHUNK * CHUNK
NP = 10240
RPT = NP // NS

_mesh = plsc.VectorSubcoreMesh(core_axis_name="c", subcore_axis_name="s")

_HIGH = lax.Precision.HIGHEST


@functools.partial(
    pl.kernel,
    out_type=jax.ShapeDtypeStruct((NC, NP, D), jnp.float32),
    mesh=_mesh,
    scratch_types=[
        pltpu.VMEM((NCHUNK, CHUNK), jnp.int32),
        pltpu.VMEM((CHUNK, D), jnp.float32),
        pltpu.VMEM_SHARED((NP, D), jnp.float32),
    ],
)
def _sc_deg(zeros_hbm, ones_hbm, dst3_hbm, out_hbm, dsts_v, ones_v, acc_sh):
    c = lax.axis_index("c")
    s = lax.axis_index("s")
    wid = s * NC + c
    row0 = s * RPT
    pltpu.sync_copy(zeros_hbm.at[pl.ds(row0, RPT)], acc_sh.at[pl.ds(row0, RPT)])
    pltpu.sync_copy(ones_hbm, ones_v)
    pltpu.sync_copy(dst3_hbm.at[wid], dsts_v)
    plsc.subcore_barrier()

    def body(i, carry):
        pltpu.sync_copy(ones_v, acc_sh.at[dsts_v.at[i]], add=True)
        return carry

    lax.fori_loop(0, NCHUNK, body, 0)
    plsc.subcore_barrier()
    pltpu.sync_copy(acc_sh.at[pl.ds(row0, RPT)], out_hbm.at[c, pl.ds(row0, RPT)])


@functools.partial(
    pl.kernel,
    out_type=jax.ShapeDtypeStruct((NC, NP, D), jnp.float32),
    mesh=_mesh,
    scratch_types=[
        pltpu.VMEM((HALF, CHUNK), jnp.int32),
        pltpu.VMEM((HALF, CHUNK), jnp.int32),
        pltpu.VMEM((CHUNK, D), jnp.float32),
        pltpu.VMEM((CHUNK, D), jnp.float32),
        pltpu.VMEM_SHARED((NP, D), jnp.float32),
        pltpu.SemaphoreType.DMA,
        pltpu.SemaphoreType.DMA,
    ],
)
def _sc_conv(zeros_hbm, hs_hbm, src3_hbm, dst3_hbm, out_hbm,
             srcs_v, dsts_v, rows_a, rows_b, acc_sh, sem_a, sem_b):
    c = lax.axis_index("c")
    s = lax.axis_index("s")
    wid = s * NC + c
    row0 = s * RPT
    pltpu.sync_copy(zeros_hbm.at[pl.ds(row0, RPT)], acc_sh.at[pl.ds(row0, RPT)])
    plsc.subcore_barrier()

    def wait_gather(i, buf, sem):
        pltpu.make_async_copy(hs_hbm.at[srcs_v.at[i]], buf, sem).wait()

    for h in range(2):
        pltpu.sync_copy(src3_hbm.at[wid, pl.ds(h * HALF, HALF)], srcs_v)
        pltpu.sync_copy(dst3_hbm.at[wid, pl.ds(h * HALF, HALF)], dsts_v)
        pltpu.async_copy(hs_hbm.at[srcs_v.at[0]], rows_a, sem_a)

        def body(g, carry):
            i0 = g * 2
            wait_gather(i0, rows_a, sem_a)
            pltpu.async_copy(hs_hbm.at[srcs_v.at[i0 + 1]], rows_b, sem_b)
            pltpu.sync_copy(rows_a, acc_sh.at[dsts_v.at[i0]], add=True)
            wait_gather(i0 + 1, rows_b, sem_b)

            @pl.when(g + 1 < HALF // 2)
            def _():
                pltpu.async_copy(hs_hbm.at[srcs_v.at[i0 + 2]], rows_a, sem_a)

            pltpu.sync_copy(rows_b, acc_sh.at[dsts_v.at[i0 + 1]], add=True)
            return carry

        lax.fori_loop(0, HALF // 2, body, 0)

    plsc.subcore_barrier()
    pltpu.sync_copy(acc_sh.at[pl.ds(row0, RPT)], out_hbm.at[c, pl.ds(row0, RPT)])


def _tc_xw_body(x_ref, w1_ref, u_ref):
    u_ref[...] = jnp.dot(x_ref[...], w1_ref[...],
                         preferred_element_type=jnp.float32, precision=_HIGH)


_tc_xw = pl.pallas_call(
    _tc_xw_body,
    out_shape=[jax.ShapeDtypeStruct((N, D), jnp.float32)],
)


def _tc_pre_body(parts_ref, u_ref, hs_ref, dis_ref):
    cnt = parts_ref[0, :, 0:1] + parts_ref[1, :, 0:1]
    dis = lax.rsqrt(cnt + 1.0)
    hs_ref[...] = u_ref[...] * dis
    dis_ref[...] = dis


_tc_pre = pl.pallas_call(
    _tc_pre_body,
    out_shape=[
        jax.ShapeDtypeStruct((N, D), jnp.float32),
        jax.ShapeDtypeStruct((N, 1), jnp.float32),
    ],
)


def _tc_mid_body(acc_ref, hs_ref, dis_ref, b_ref, w2_ref, hs2_ref):
    tot = acc_ref[0] + acc_ref[1] + hs_ref[...]
    h1 = jnp.maximum(dis_ref[...] * tot + b_ref[...], 0.0)
    hs2_ref[...] = jnp.dot(h1, w2_ref[...],
                           preferred_element_type=jnp.float32,
                           precision=_HIGH) * dis_ref[...]


_tc_mid = pl.pallas_call(
    _tc_mid_body,
    out_shape=[jax.ShapeDtypeStruct((N, H), jnp.float32)],
)


def _tc_post_body(acc_ref, hs2_ref, dis_ref, b2_ref, attw_ref, attb_ref,
                  batch_ref, projw_ref, projb_ref, c1w_ref, c1b_ref,
                  c2w_ref, c2b_ref, out_ref):
    tot = acc_ref[0] + acc_ref[1] + hs2_ref[...]
    h2 = jnp.maximum(dis_ref[...] * tot + b2_ref[...], 0.0)
    z = jnp.dot(h2, attw_ref[...], preferred_element_type=jnp.float32,
                precision=_HIGH) + attb_ref[...]
    att = jnp.where(z > 0, z, 0.01 * z)
    b = batch_ref[...]
    gid = lax.broadcasted_iota(jnp.int32, (N, G), 1)
    mask = b == gid
    onehot = mask.astype(jnp.float32)
    segmax = jnp.max(jnp.where(mask, att, -1e30), axis=0, keepdims=True)
    maxn = lax.dot_general(onehot, segmax, (((1,), (1,)), ((), ())),
                           preferred_element_type=jnp.float32,
                           precision=_HIGH)
    e = jnp.exp(att - maxn)
    denom = lax.dot_general(onehot, e, (((0,), (0,)), ((), ())),
                            preferred_element_type=jnp.float32,
                            precision=_HIGH)
    num = lax.dot_general(onehot, e * h2, (((0,), (0,)), ((), ())),
                          preferred_element_type=jnp.float32,
                          precision=_HIGH)
    g = num / (denom + 1e-16)
    p = jnp.dot(g, projw_ref[...], preferred_element_type=jnp.float32,
                precision=_HIGH) + projb_ref[...]
    q = jnp.maximum(jnp.dot(p, c1w_ref[...],
                            preferred_element_type=jnp.float32,
                            precision=_HIGH) + c1b_ref[...], 0.0)
    out_ref[...] = jnp.dot(q, c2w_ref[...],
                           preferred_element_type=jnp.float32,
                           precision=_HIGH) + c2b_ref[...]


_tc_post = pl.pallas_call(
    _tc_post_body,
    out_shape=[jax.ShapeDtypeStruct((G, C), jnp.float32)],
)


def kernel(x, edge_index, batch, W1, b1, W2, b2, attW, attb,
           projW, projb, c1W, c1b, c2W, c2b):
    pad = EP - E
    pidx = jnp.arange(pad, dtype=jnp.int32)
    src3 = jnp.concatenate([edge_index[0], pidx % N]
                           ).reshape(NW, NCHUNK, CHUNK)
    dst3 = jnp.concatenate([edge_index[1], N + pidx % (NP - N)]
                           ).reshape(NW, NCHUNK, CHUNK)
    zeros_nd = jnp.zeros((NP, D), jnp.float32)
    ones_cd = jnp.ones((CHUNK, D), jnp.float32)

    (u,) = _tc_xw(x, W1)
    deg_parts = _sc_deg(zeros_nd, ones_cd, dst3)[:, :N, :]
    hs1, dis = _tc_pre(deg_parts, u)
    acc1 = _sc_conv(zeros_nd, hs1, src3, dst3)[:, :N, :]
    (hs2,) = _tc_mid(acc1, hs1, dis, b1.reshape(1, H), W2)
    acc2 = _sc_conv(zeros_nd, hs2, src3, dst3)[:, :N, :]
    (out,) = _tc_post(acc2, hs2, dis, b2.reshape(1, H), attW,
                      attb.reshape(1, 1), batch.reshape(N, 1), projW,
                      projb.reshape(1, EMB), c1W, c1b.reshape(1, 128),
                      c2W, c2b.reshape(1, C))
    return out


# speedup vs baseline: 1.0328x; 1.0328x over previous
import functools

import jax
import jax.numpy as jnp
from jax import lax
from jax.experimental import pallas as pl
from jax.experimental.pallas import tpu as pltpu
from jax.experimental.pallas import tpu_sc as plsc

N = 10000
E = 320000
D = 128
H = 128
G = 256
EMB = 300
C = 10

NC = 2
NS = 16
NW = NC * NS
CHUNK = 96
NCHUNK = 112
NSEC = 7
SEC = NCHUNK // NSEC
EP = NW * NCHUNK * CHUNK
EPT = NCHUNK * CHUNK
NP = 10240
RPT = NP // NS

_mesh = plsc.VectorSubcoreMesh(core_axis_name="c", subcore_axis_name="s")

_HIGH = lax.Precision.HIGHEST


@functools.partial(
    pl.kernel,
    out_type=jax.ShapeDtypeStruct((NC, NP, D), jnp.float32),
    mesh=_mesh,
    scratch_types=[
        pltpu.VMEM((NCHUNK, CHUNK), jnp.int32),
        pltpu.VMEM((CHUNK, D), jnp.float32),
        pltpu.VMEM_SHARED((NP, D), jnp.float32),
    ],
)
def _sc_deg(zeros_hbm, ones_hbm, dst3_hbm, out_hbm, dsts_v, ones_v, acc_sh):
    c = lax.axis_index("c")
    s = lax.axis_index("s")
    wid = s * NC + c
    row0 = s * RPT
    pltpu.sync_copy(zeros_hbm.at[pl.ds(row0, RPT)], acc_sh.at[pl.ds(row0, RPT)])
    pltpu.sync_copy(ones_hbm, ones_v)
    pltpu.sync_copy(dst3_hbm.at[wid], dsts_v)
    plsc.subcore_barrier()

    def body(i, carry):
        pltpu.sync_copy(ones_v, acc_sh.at[dsts_v.at[i]], add=True)
        return carry

    lax.fori_loop(0, NCHUNK, body, 0)
    plsc.subcore_barrier()
    pltpu.sync_copy(acc_sh.at[pl.ds(row0, RPT)], out_hbm.at[c, pl.ds(row0, RPT)])


@functools.partial(
    pl.kernel,
    out_type=jax.ShapeDtypeStruct((NC, NP, D), jnp.float32),
    mesh=_mesh,
    scratch_types=[
        pltpu.VMEM((SEC, CHUNK), jnp.int32),
        pltpu.VMEM((SEC, CHUNK), jnp.int32),
        pltpu.VMEM((CHUNK, D), jnp.float32),
        pltpu.VMEM((CHUNK, D), jnp.float32),
        pltpu.VMEM((CHUNK, D), jnp.float32),
        pltpu.VMEM_SHARED((NP, D), jnp.float32),
        pltpu.SemaphoreType.DMA,
        pltpu.SemaphoreType.DMA,
        pltpu.SemaphoreType.DMA,
        pltpu.SemaphoreType.DMA,
        pltpu.SemaphoreType.DMA,
        pltpu.SemaphoreType.DMA,
    ],
)
def _sc_conv(zeros_hbm, hs_hbm, src3_hbm, dst3_hbm, out_hbm,
             srcs_v, dsts_v, rows_a, rows_b, rows_c, acc_sh,
             gs_a, gs_b, gs_c, ss_a, ss_b, ss_c):
    c = lax.axis_index("c")
    s = lax.axis_index("s")
    wid = s * NC + c
    row0 = s * RPT
    pltpu.sync_copy(zeros_hbm.at[pl.ds(row0, RPT)], acc_sh.at[pl.ds(row0, RPT)])
    plsc.subcore_barrier()

    rows = (rows_a, rows_b, rows_c)
    gsem = (gs_a, gs_b, gs_c)
    ssem = (ss_a, ss_b, ss_c)

    def wait_gather(i, b):
        pltpu.make_async_copy(hs_hbm.at[srcs_v.at[i]], rows[b], gsem[b]).wait()

    def wait_scatter(i, b):
        pltpu.make_async_copy(rows[b], acc_sh.at[dsts_v.at[i]],
                              ssem[b]).wait()

    for h in range(NSEC):
        pltpu.sync_copy(src3_hbm.at[wid, pl.ds(h * SEC, SEC)], srcs_v)
        pltpu.sync_copy(dst3_hbm.at[wid, pl.ds(h * SEC, SEC)], dsts_v)
        pltpu.async_copy(hs_hbm.at[srcs_v.at[0]], rows[0], gsem[0])
        pltpu.async_copy(hs_hbm.at[srcs_v.at[1]], rows[1], gsem[1])

        def body(g, carry):
            i0 = g * 3
            for k in range(3):
                i = i0 + k
                b = k
                nb = (k + 2) % 3
                wait_gather(i, b)
                pltpu.async_copy(rows[b], acc_sh.at[dsts_v.at[i]],
                                 ssem[b], add=True)

                @pl.when(i + 2 < SEC)
                def _():
                    @pl.when(g + k > 0)
                    def _():
                        wait_scatter(i - 1, nb)
                    pltpu.async_copy(hs_hbm.at[srcs_v.at[i + 2]],
                                     rows[nb], gsem[nb])

            return carry

        lax.fori_loop(0, SEC // 3, body, 0)
        wait_gather(SEC - 1, (SEC - 1) % 3)
        pltpu.async_copy(rows[(SEC - 1) % 3],
                         acc_sh.at[dsts_v.at[SEC - 1]],
                         ssem[(SEC - 1) % 3], add=True)
        wait_scatter(SEC - 3, (SEC - 3) % 3)
        wait_scatter(SEC - 2, (SEC - 2) % 3)
        wait_scatter(SEC - 1, (SEC - 1) % 3)

    plsc.subcore_barrier()
    pltpu.sync_copy(acc_sh.at[pl.ds(row0, RPT)], out_hbm.at[c, pl.ds(row0, RPT)])


def _tc_pre_body(parts_ref, x_ref, w1_ref, hs_ref, dis_ref):
    cnt = parts_ref[0, :, 0:1] + parts_ref[1, :, 0:1]
    dis = lax.rsqrt(cnt + 1.0)
    h = jnp.dot(x_ref[...], w1_ref[...],
                preferred_element_type=jnp.float32, precision=_HIGH)
    hs_ref[...] = h * dis
    dis_ref[...] = dis


_tc_pre = pl.pallas_call(
    _tc_pre_body,
    out_shape=[
        jax.ShapeDtypeStruct((N, D), jnp.float32),
        jax.ShapeDtypeStruct((N, 1), jnp.float32),
    ],
)


def _tc_mid_body(acc_ref, hs_ref, dis_ref, b_ref, w2_ref, hs2_ref):
    tot = acc_ref[0] + acc_ref[1] + hs_ref[...]
    h1 = jnp.maximum(dis_ref[...] * tot + b_ref[...], 0.0)
    hs2_ref[...] = jnp.dot(h1, w2_ref[...],
                           preferred_element_type=jnp.float32,
                           precision=_HIGH) * dis_ref[...]


_tc_mid = pl.pallas_call(
    _tc_mid_body,
    out_shape=[jax.ShapeDtypeStruct((N, H), jnp.float32)],
)


def _tc_post_body(acc_ref, hs2_ref, dis_ref, b2_ref, attw_ref, attb_ref,
                  batch_ref, projw_ref, projb_ref, c1w_ref, c1b_ref,
                  c2w_ref, c2b_ref, out_ref):
    tot = acc_ref[0] + acc_ref[1] + hs2_ref[...]
    h2 = jnp.maximum(dis_ref[...] * tot + b2_ref[...], 0.0)
    z = jnp.dot(h2, attw_ref[...], preferred_element_type=jnp.float32,
                precision=_HIGH) + attb_ref[...]
    att = jnp.where(z > 0, z, 0.01 * z)
    b = batch_ref[...]
    gid = lax.broadcasted_iota(jnp.int32, (N, G), 1)
    mask = b == gid
    onehot = mask.astype(jnp.float32)
    segmax = jnp.max(jnp.where(mask, att, -1e30), axis=0, keepdims=True)
    maxn = lax.dot_general(onehot, segmax, (((1,), (1,)), ((), ())),
                           preferred_element_type=jnp.float32,
                           precision=_HIGH)
    e = jnp.exp(att - maxn)
    denom = lax.dot_general(onehot, e, (((0,), (0,)), ((), ())),
                            preferred_element_type=jnp.float32,
                            precision=_HIGH)
    num = lax.dot_general(onehot, e * h2, (((0,), (0,)), ((), ())),
                          preferred_element_type=jnp.float32,
                          precision=_HIGH)
    g = num / (denom + 1e-16)
    p = jnp.dot(g, projw_ref[...], preferred_element_type=jnp.float32,
                precision=_HIGH) + projb_ref[...]
    q = jnp.maximum(jnp.dot(p, c1w_ref[...],
                            preferred_element_type=jnp.float32,
                            precision=_HIGH) + c1b_ref[...], 0.0)
    out_ref[...] = jnp.dot(q, c2w_ref[...],
                           preferred_element_type=jnp.float32,
                           precision=_HIGH) + c2b_ref[...]


_tc_post = pl.pallas_call(
    _tc_post_body,
    out_shape=[jax.ShapeDtypeStruct((G, C), jnp.float32)],
)


def kernel(x, edge_index, batch, W1, b1, W2, b2, attW, attb,
           projW, projb, c1W, c1b, c2W, c2b):
    pad = EP - E
    pidx = jnp.arange(pad, dtype=jnp.int32)
    src3 = jnp.concatenate([edge_index[0], pidx % N]
                           ).reshape(NW, NCHUNK, CHUNK)
    dst3 = jnp.concatenate([edge_index[1], N + pidx % (NP - N)]
                           ).reshape(NW, NCHUNK, CHUNK)
    zeros_nd = jnp.zeros((NP, D), jnp.float32)
    ones_cd = jnp.ones((CHUNK, D), jnp.float32)

    deg_parts = _sc_deg(zeros_nd, ones_cd, dst3)[:, :N, :]
    hs1, dis = _tc_pre(deg_parts, x, W1)
    acc1 = _sc_conv(zeros_nd, hs1, src3, dst3)[:, :N, :]
    (hs2,) = _tc_mid(acc1, hs1, dis, b1.reshape(1, H), W2)
    acc2 = _sc_conv(zeros_nd, hs2, src3, dst3)[:, :N, :]
    (out,) = _tc_post(acc2, hs2, dis, b2.reshape(1, H), attW,
                      attb.reshape(1, 1), batch.reshape(N, 1), projW,
                      projb.reshape(1, EMB), c1W, c1b.reshape(1, 128),
                      c2W, c2b.reshape(1, C))
    return out


# speedup vs baseline: 1.0355x; 1.0026x over previous
import functools

import jax
import jax.numpy as jnp
from jax import lax
from jax.experimental import pallas as pl
from jax.experimental.pallas import tpu as pltpu
from jax.experimental.pallas import tpu_sc as plsc

N = 10000
E = 320000
D = 128
H = 128
G = 256
EMB = 300
C = 10

NC = 2
NS = 16
NW = NC * NS
CHUNK = 96
NCHUNK = 112
NSEC = 7
SEC = NCHUNK // NSEC
EP = NW * NCHUNK * CHUNK
EPT = NCHUNK * CHUNK
NP = 10240
RPT = NP // NS

_mesh = plsc.VectorSubcoreMesh(core_axis_name="c", subcore_axis_name="s")

_HIGH = lax.Precision.HIGHEST


@functools.partial(
    pl.kernel,
    out_type=jax.ShapeDtypeStruct((NC, NP, D), jnp.float32),
    mesh=_mesh,
    scratch_types=[
        pltpu.VMEM((NCHUNK, CHUNK), jnp.int32),
        pltpu.VMEM((CHUNK, D), jnp.float32),
        pltpu.VMEM_SHARED((NP, D), jnp.float32),
        pltpu.SemaphoreType.DMA,
    ],
)
def _sc_deg(zeros_hbm, ones_hbm, dst3_hbm, out_hbm, dsts_v, ones_v, acc_sh,
            sem):
    c = lax.axis_index("c")
    s = lax.axis_index("s")
    wid = s * NC + c
    row0 = s * RPT
    pltpu.sync_copy(zeros_hbm.at[pl.ds(row0, RPT)], acc_sh.at[pl.ds(row0, RPT)])
    pltpu.sync_copy(ones_hbm, ones_v)
    pltpu.sync_copy(dst3_hbm.at[wid], dsts_v)
    plsc.subcore_barrier()

    def body(g, carry):
        i0 = g * 8
        for k in range(8):
            pltpu.async_copy(ones_v, acc_sh.at[dsts_v.at[i0 + k]], sem,
                             add=True)
        for k in range(8):
            pltpu.make_async_copy(ones_v, acc_sh.at[dsts_v.at[i0 + k]],
                                  sem).wait()
        return carry

    lax.fori_loop(0, NCHUNK // 8, body, 0)
    plsc.subcore_barrier()
    pltpu.sync_copy(acc_sh.at[pl.ds(row0, RPT)], out_hbm.at[c, pl.ds(row0, RPT)])


@functools.partial(
    pl.kernel,
    out_type=jax.ShapeDtypeStruct((NC, NP, D), jnp.float32),
    mesh=_mesh,
    scratch_types=[
        pltpu.VMEM((SEC, CHUNK), jnp.int32),
        pltpu.VMEM((SEC, CHUNK), jnp.int32),
        pltpu.VMEM((CHUNK, D), jnp.float32),
        pltpu.VMEM((CHUNK, D), jnp.float32),
        pltpu.VMEM((CHUNK, D), jnp.float32),
        pltpu.VMEM_SHARED((NP, D), jnp.float32),
        pltpu.SemaphoreType.DMA,
        pltpu.SemaphoreType.DMA,
        pltpu.SemaphoreType.DMA,
        pltpu.SemaphoreType.DMA,
        pltpu.SemaphoreType.DMA,
        pltpu.SemaphoreType.DMA,
    ],
)
def _sc_conv(zeros_hbm, hs_hbm, src3_hbm, dst3_hbm, out_hbm,
             srcs_v, dsts_v, rows_a, rows_b, rows_c, acc_sh,
             gs_a, gs_b, gs_c, ss_a, ss_b, ss_c):
    c = lax.axis_index("c")
    s = lax.axis_index("s")
    wid = s * NC + c
    row0 = s * RPT
    pltpu.sync_copy(zeros_hbm.at[pl.ds(row0, RPT)], acc_sh.at[pl.ds(row0, RPT)])
    plsc.subcore_barrier()

    rows = (rows_a, rows_b, rows_c)
    gsem = (gs_a, gs_b, gs_c)
    ssem = (ss_a, ss_b, ss_c)

    def wait_gather(i, b):
        pltpu.make_async_copy(hs_hbm.at[srcs_v.at[i]], rows[b], gsem[b]).wait()

    def wait_scatter(i, b):
        pltpu.make_async_copy(rows[b], acc_sh.at[dsts_v.at[i]],
                              ssem[b]).wait()

    for h in range(NSEC):
        pltpu.sync_copy(src3_hbm.at[wid, pl.ds(h * SEC, SEC)], srcs_v)
        pltpu.sync_copy(dst3_hbm.at[wid, pl.ds(h * SEC, SEC)], dsts_v)
        pltpu.async_copy(hs_hbm.at[srcs_v.at[0]], rows[0], gsem[0])
        pltpu.async_copy(hs_hbm.at[srcs_v.at[1]], rows[1], gsem[1])

        def body(g, carry):
            i0 = g * 3
            for k in range(3):
                i = i0 + k
                b = k
                nb = (k + 2) % 3
                wait_gather(i, b)
                pltpu.async_copy(rows[b], acc_sh.at[dsts_v.at[i]],
                                 ssem[b], add=True)

                @pl.when(i + 2 < SEC)
                def _():
                    @pl.when(g + k > 0)
                    def _():
                        wait_scatter(i - 1, nb)
                    pltpu.async_copy(hs_hbm.at[srcs_v.at[i + 2]],
                                     rows[nb], gsem[nb])

            return carry

        lax.fori_loop(0, SEC // 3, body, 0)
        wait_gather(SEC - 1, (SEC - 1) % 3)
        pltpu.async_copy(rows[(SEC - 1) % 3],
                         acc_sh.at[dsts_v.at[SEC - 1]],
                         ssem[(SEC - 1) % 3], add=True)
        wait_scatter(SEC - 3, (SEC - 3) % 3)
        wait_scatter(SEC - 2, (SEC - 2) % 3)
        wait_scatter(SEC - 1, (SEC - 1) % 3)

    plsc.subcore_barrier()
    pltpu.sync_copy(acc_sh.at[pl.ds(row0, RPT)], out_hbm.at[c, pl.ds(row0, RPT)])


def _tc_pre_body(parts_ref, x_ref, w1_ref, hs_ref, dis_ref):
    cnt = parts_ref[0, :, 0:1] + parts_ref[1, :, 0:1]
    dis = lax.rsqrt(cnt + 1.0)
    h = jnp.dot(x_ref[...], w1_ref[...],
                preferred_element_type=jnp.float32, precision=_HIGH)
    hs_ref[...] = h * dis
    dis_ref[...] = dis


_tc_pre = pl.pallas_call(
    _tc_pre_body,
    out_shape=[
        jax.ShapeDtypeStruct((N, D), jnp.float32),
        jax.ShapeDtypeStruct((N, 1), jnp.float32),
    ],
)


def _tc_mid_body(acc_ref, hs_ref, dis_ref, b_ref, w2_ref, hs2_ref):
    tot = acc_ref[0] + acc_ref[1] + hs_ref[...]
    h1 = jnp.maximum(dis_ref[...] * tot + b_ref[...], 0.0)
    hs2_ref[...] = jnp.dot(h1, w2_ref[...],
                           preferred_element_type=jnp.float32,
                           precision=_HIGH) * dis_ref[...]


_tc_mid = pl.pallas_call(
    _tc_mid_body,
    out_shape=[jax.ShapeDtypeStruct((N, H), jnp.float32)],
)


def _tc_post_body(acc_ref, hs2_ref, dis_ref, b2_ref, attw_ref, attb_ref,
                  batch_ref, projw_ref, projb_ref, c1w_ref, c1b_ref,
                  c2w_ref, c2b_ref, out_ref):
    tot = acc_ref[0] + acc_ref[1] + hs2_ref[...]
    h2 = jnp.maximum(dis_ref[...] * tot + b2_ref[...], 0.0)
    z = jnp.dot(h2, attw_ref[...], preferred_element_type=jnp.float32,
                precision=_HIGH) + attb_ref[...]
    att = jnp.where(z > 0, z, 0.01 * z)
    b = batch_ref[...]
    gid = lax.broadcasted_iota(jnp.int32, (N, G), 1)
    mask = b == gid
    onehot = mask.astype(jnp.float32)
    segmax = jnp.max(jnp.where(mask, att, -1e30), axis=0, keepdims=True)
    maxn = lax.dot_general(onehot, segmax, (((1,), (1,)), ((), ())),
                           preferred_element_type=jnp.float32,
                           precision=_HIGH)
    e = jnp.exp(att - maxn)
    denom = lax.dot_general(onehot, e, (((0,), (0,)), ((), ())),
                            preferred_element_type=jnp.float32,
                            precision=_HIGH)
    num = lax.dot_general(onehot, e * h2, (((0,), (0,)), ((), ())),
                          preferred_element_type=jnp.float32,
                          precision=_HIGH)
    g = num / (denom + 1e-16)
    p = jnp.dot(g, projw_ref[...], preferred_element_type=jnp.float32,
                precision=_HIGH) + projb_ref[...]
    q = jnp.maximum(jnp.dot(p, c1w_ref[...],
                            preferred_element_type=jnp.float32,
                            precision=_HIGH) + c1b_ref[...], 0.0)
    out_ref[...] = jnp.dot(q, c2w_ref[...],
                           preferred_element_type=jnp.float32,
                           precision=_HIGH) + c2b_ref[...]


_tc_post = pl.pallas_call(
    _tc_post_body,
    out_shape=[jax.ShapeDtypeStruct((G, C), jnp.float32)],
)


def kernel(x, edge_index, batch, W1, b1, W2, b2, attW, attb,
           projW, projb, c1W, c1b, c2W, c2b):
    pad = EP - E
    pidx = jnp.arange(pad, dtype=jnp.int32)
    src3 = jnp.concatenate([edge_index[0], pidx % N]
                           ).reshape(NW, NCHUNK, CHUNK)
    dst3 = jnp.concatenate([edge_index[1], N + pidx % (NP - N)]
                           ).reshape(NW, NCHUNK, CHUNK)
    zeros_nd = jnp.zeros((NP, D), jnp.float32)
    ones_cd = jnp.ones((CHUNK, D), jnp.float32)

    deg_parts = _sc_deg(zeros_nd, ones_cd, dst3)[:, :N, :]
    hs1, dis = _tc_pre(deg_parts, x, W1)
    acc1 = _sc_conv(zeros_nd, hs1, src3, dst3)[:, :N, :]
    (hs2,) = _tc_mid(acc1, hs1, dis, b1.reshape(1, H), W2)
    acc2 = _sc_conv(zeros_nd, hs2, src3, dst3)[:, :N, :]
    (out,) = _tc_post(acc2, hs2, dis, b2.reshape(1, H), attW,
                      attb.reshape(1, 1), batch.reshape(N, 1), projW,
                      projb.reshape(1, EMB), c1W, c1b.reshape(1, 128),
                      c2W, c2b.reshape(1, C))
    return out


# speedup vs baseline: 1.0836x; 1.0465x over previous
import functools

import jax
import jax.numpy as jnp
from jax import lax
from jax.experimental import pallas as pl
from jax.experimental.pallas import tpu as pltpu
from jax.experimental.pallas import tpu_sc as plsc

N = 10000
E = 320000
D = 128
H = 128
G = 256
EMB = 300
C = 10

NC = 2
NS = 16
NW = NC * NS
CHUNK = 96
NCHUNK = 112
NSEC = 7
SEC = NCHUNK // NSEC
EP = NW * NCHUNK * CHUNK
EPT = NCHUNK * CHUNK
NP = 10240
RPT = NP // NS

_mesh = plsc.VectorSubcoreMesh(core_axis_name="c", subcore_axis_name="s")

_HIGH = lax.Precision.HIGHEST


@functools.partial(
    pl.kernel,
    out_type=jax.ShapeDtypeStruct((NC, NP, D), jnp.float32),
    mesh=_mesh,
    scratch_types=[
        pltpu.VMEM((NCHUNK, CHUNK), jnp.int32),
        pltpu.VMEM((CHUNK, D), jnp.float32),
        pltpu.VMEM_SHARED((NP, D), jnp.float32),
        pltpu.SemaphoreType.DMA,
    ],
)
def _sc_deg(zeros_hbm, ones_hbm, dst3_hbm, out_hbm, dsts_v, ones_v, acc_sh,
            sem):
    c = lax.axis_index("c")
    s = lax.axis_index("s")
    wid = s * NC + c
    row0 = s * RPT
    pltpu.sync_copy(zeros_hbm.at[pl.ds(row0, RPT)], acc_sh.at[pl.ds(row0, RPT)])
    pltpu.sync_copy(ones_hbm, ones_v)
    pltpu.sync_copy(dst3_hbm.at[wid], dsts_v)
    plsc.subcore_barrier()

    def body(g, carry):
        i0 = g * 8
        for k in range(8):
            pltpu.async_copy(ones_v, acc_sh.at[dsts_v.at[i0 + k]], sem,
                             add=True)
        for k in range(8):
            pltpu.make_async_copy(ones_v, acc_sh.at[dsts_v.at[i0 + k]],
                                  sem).wait()
        return carry

    lax.fori_loop(0, NCHUNK // 8, body, 0)
    plsc.subcore_barrier()
    pltpu.sync_copy(acc_sh.at[pl.ds(row0, RPT)], out_hbm.at[c, pl.ds(row0, RPT)])


@functools.partial(
    pl.kernel,
    out_type=jax.ShapeDtypeStruct((NC, NP, D), jnp.float32),
    mesh=_mesh,
    scratch_types=[
        pltpu.VMEM((SEC, CHUNK), jnp.int32),
        pltpu.VMEM((SEC, CHUNK), jnp.int32),
        pltpu.VMEM((CHUNK, D), jnp.float32),
        pltpu.VMEM((CHUNK, D), jnp.float32),
        pltpu.VMEM((CHUNK, D), jnp.float32),
        pltpu.VMEM_SHARED((NP, D), jnp.float32),
        pltpu.SemaphoreType.DMA,
        pltpu.SemaphoreType.DMA,
        pltpu.SemaphoreType.DMA,
        pltpu.SemaphoreType.DMA,
        pltpu.SemaphoreType.DMA,
        pltpu.SemaphoreType.DMA,
    ],
)
def _sc_conv(zeros_hbm, hs_hbm, src3_hbm, dst3_hbm, out_hbm,
             srcs_v, dsts_v, rows_a, rows_b, rows_c, acc_sh,
             gs_a, gs_b, gs_c, ss_a, ss_b, ss_c):
    c = lax.axis_index("c")
    s = lax.axis_index("s")
    wid = s * NC + c
    row0 = s * RPT
    pltpu.sync_copy(zeros_hbm.at[pl.ds(row0, RPT)], acc_sh.at[pl.ds(row0, RPT)])
    plsc.subcore_barrier()

    rows = (rows_a, rows_b, rows_c)
    gsem = (gs_a, gs_b, gs_c)
    ssem = (ss_a, ss_b, ss_c)

    def wait_gather(i, b):
        pltpu.make_async_copy(hs_hbm.at[srcs_v.at[i]], rows[b], gsem[b]).wait()

    def wait_scatter(i, b):
        pltpu.make_async_copy(rows[b], acc_sh.at[dsts_v.at[i]],
                              ssem[b]).wait()

    for h in range(NSEC):
        pltpu.sync_copy(src3_hbm.at[wid, pl.ds(h * SEC, SEC)], srcs_v)
        pltpu.sync_copy(dst3_hbm.at[wid, pl.ds(h * SEC, SEC)], dsts_v)
        pltpu.async_copy(hs_hbm.at[srcs_v.at[0]], rows[0], gsem[0])
        pltpu.async_copy(hs_hbm.at[srcs_v.at[1]], rows[1], gsem[1])

        def body(g, carry):
            i0 = g * 3
            for k in range(3):
                i = i0 + k
                b = k
                nb = (k + 2) % 3
                wait_gather(i, b)
                pltpu.async_copy(rows[b], acc_sh.at[dsts_v.at[i]],
                                 ssem[b], add=True)

                @pl.when(i + 2 < SEC)
                def _():
                    @pl.when(g + k > 0)
                    def _():
                        wait_scatter(i - 1, nb)
                    pltpu.async_copy(hs_hbm.at[srcs_v.at[i + 2]],
                                     rows[nb], gsem[nb])

            return carry

        lax.fori_loop(0, SEC // 3, body, 0)
        wait_gather(SEC - 1, (SEC - 1) % 3)
        pltpu.async_copy(rows[(SEC - 1) % 3],
                         acc_sh.at[dsts_v.at[SEC - 1]],
                         ssem[(SEC - 1) % 3], add=True)
        wait_scatter(SEC - 3, (SEC - 3) % 3)
        wait_scatter(SEC - 2, (SEC - 2) % 3)
        wait_scatter(SEC - 1, (SEC - 1) % 3)

    plsc.subcore_barrier()
    pltpu.sync_copy(acc_sh.at[pl.ds(row0, RPT)], out_hbm.at[c, pl.ds(row0, RPT)])


def _tc_pre_body(parts_ref, x_ref, w1_ref, hs_ref, dis_ref):
    cnt = parts_ref[0, :N, 0:1] + parts_ref[1, :N, 0:1]
    dis = lax.rsqrt(cnt + 1.0)
    h = jnp.dot(x_ref[...], w1_ref[...],
                preferred_element_type=jnp.float32, precision=_HIGH)
    hs_ref[...] = h * dis
    dis_ref[...] = dis


_tc_pre = pl.pallas_call(
    _tc_pre_body,
    out_shape=[
        jax.ShapeDtypeStruct((N, D), jnp.float32),
        jax.ShapeDtypeStruct((N, 1), jnp.float32),
    ],
)


def _tc_mid_body(acc_ref, hs_ref, dis_ref, b_ref, w2_ref, hs2_ref):
    tot = acc_ref[0, :N] + acc_ref[1, :N] + hs_ref[...]
    h1 = jnp.maximum(dis_ref[...] * tot + b_ref[...], 0.0)
    hs2_ref[...] = jnp.dot(h1, w2_ref[...],
                           preferred_element_type=jnp.float32,
                           precision=_HIGH) * dis_ref[...]


_tc_mid = pl.pallas_call(
    _tc_mid_body,
    out_shape=[jax.ShapeDtypeStruct((N, H), jnp.float32)],
)


def _tc_post_body(acc_ref, hs2_ref, dis_ref, b2_ref, attw_ref, attb_ref,
                  batch_ref, projw_ref, projb_ref, c1w_ref, c1b_ref,
                  c2w_ref, c2b_ref, out_ref):
    tot = acc_ref[0, :N] + acc_ref[1, :N] + hs2_ref[...]
    h2 = jnp.maximum(dis_ref[...] * tot + b2_ref[...], 0.0)
    z = jnp.dot(h2, attw_ref[...], preferred_element_type=jnp.float32,
                precision=_HIGH) + attb_ref[...]
    att = jnp.where(z > 0, z, 0.01 * z)
    b = batch_ref[...]
    gid = lax.broadcasted_iota(jnp.int32, (N, G), 1)
    mask = b == gid
    onehot = mask.astype(jnp.float32)
    segmax = jnp.max(jnp.where(mask, att, -1e30), axis=0, keepdims=True)
    maxn = lax.dot_general(onehot, segmax, (((1,), (1,)), ((), ())),
                           preferred_element_type=jnp.float32,
                           precision=_HIGH)
    e = jnp.exp(att - maxn)
    denom = lax.dot_general(onehot, e, (((0,), (0,)), ((), ())),
                            preferred_element_type=jnp.float32,
                            precision=_HIGH)
    num = lax.dot_general(onehot, e * h2, (((0,), (0,)), ((), ())),
                          preferred_element_type=jnp.float32,
                          precision=_HIGH)
    g = num / (denom + 1e-16)
    p = jnp.dot(g, projw_ref[...], preferred_element_type=jnp.float32,
                precision=_HIGH) + projb_ref[...]
    q = jnp.maximum(jnp.dot(p, c1w_ref[...],
                            preferred_element_type=jnp.float32,
                            precision=_HIGH) + c1b_ref[...], 0.0)
    out_ref[...] = jnp.dot(q, c2w_ref[...],
                           preferred_element_type=jnp.float32,
                           precision=_HIGH) + c2b_ref[...]


_tc_post = pl.pallas_call(
    _tc_post_body,
    out_shape=[jax.ShapeDtypeStruct((G, C), jnp.float32)],
)


def kernel(x, edge_index, batch, W1, b1, W2, b2, attW, attb,
           projW, projb, c1W, c1b, c2W, c2b):
    pad = EP - E
    pidx = jnp.arange(pad, dtype=jnp.int32)
    src3 = jnp.concatenate([edge_index[0], pidx % N]
                           ).reshape(NW, NCHUNK, CHUNK)
    dst3 = jnp.concatenate([edge_index[1], N + pidx % (NP - N)]
                           ).reshape(NW, NCHUNK, CHUNK)
    zeros_nd = jnp.zeros((NP, D), jnp.float32)
    ones_cd = jnp.ones((CHUNK, D), jnp.float32)

    deg_parts = _sc_deg(zeros_nd, ones_cd, dst3)
    hs1, dis = _tc_pre(deg_parts, x, W1)
    acc1 = _sc_conv(zeros_nd, hs1, src3, dst3)
    (hs2,) = _tc_mid(acc1, hs1, dis, b1.reshape(1, H), W2)
    acc2 = _sc_conv(zeros_nd, hs2, src3, dst3)
    (out,) = _tc_post(acc2, hs2, dis, b2.reshape(1, H), attW,
                      attb.reshape(1, 1), batch.reshape(N, 1), projW,
                      projb.reshape(1, EMB), c1W, c1b.reshape(1, 128),
                      c2W, c2b.reshape(1, C))
    return out
